# baseline (speedup 1.0000x reference)
import functools

import numpy as np
import jax
import jax.numpy as jnp
from jax import lax
from jax.experimental import pallas as pl
from jax.experimental.pallas import tpu as pltpu
from jax.experimental.pallas import tpu_sc as plsc

_D = 768
_B = 4
_S = 4096
_N = _B * _S
_NC = 2
_NS = 16
_NW = _NC * _NS
_SPW = _S // _NW
_NPW = _B * _SPW
_C = 32
_CPB = _SPW // _C
_NCH = _B * _CPB
_LANES = 16
_KBLK = 4
_NKB = _D // (_LANES * _KBLK)


def _build_pe(seq_len, d_model):
    pos = np.arange(seq_len, dtype=np.float32)[:, None]
    div = np.exp(np.arange(0, d_model, 2, dtype=np.float32)
                 * (-np.log(10000.0) / d_model))
    pe = np.zeros((seq_len, d_model), dtype=np.float32)
    pe[:, 0::2] = np.sin(pos * div)
    pe[:, 1::2] = np.cos(pos * div)
    return pe


def _pack_pe_bf16(pe):
    u = pe.view(np.uint32)
    rb = ((u.astype(np.uint64) + 0x7FFF + ((u >> 16) & 1)) >> 16).astype(np.uint32)
    g = rb.reshape(pe.shape[0], pe.shape[1] // 32, 2, 16)
    words = g[:, :, 0, :] | (g[:, :, 1, :] << 16)
    return words.reshape(pe.shape[0], pe.shape[1] // 2).view(np.int32)


_PE = _build_pe(_S, _D)
_PE_PACKED = _pack_pe_bf16(_PE)
_DP = _D // 2

_mesh = plsc.VectorSubcoreMesh(core_axis_name="c", subcore_axis_name="s")


@functools.partial(
    pl.kernel,
    mesh=_mesh,
    out_type=jax.ShapeDtypeStruct((_N, _D), jnp.float32),
    scratch_types=[
        pltpu.VMEM((_NPW,), jnp.int32),
        pltpu.VMEM((_NPW * _LANES,), jnp.int32),
        pltpu.VMEM((3, _D), jnp.float32),
        pltpu.VMEM((_SPW * _DP,), jnp.int32),
        pltpu.VMEM((_C, _D), jnp.float32),
        pltpu.VMEM((_C, _D), jnp.float32),
        pltpu.SemaphoreType.DMA,
        pltpu.SemaphoreType.DMA,
        pltpu.SemaphoreType.DMA,
        pltpu.SemaphoreType.DMA,
    ],
)
def _embed(tok_hbm, seg_hbm, seq_hbm, sidrep_hbm, pe_hbm, out_hbm,
           seqv, sidrv, segtab, pebv, tok0, tok1,
           sem_t0, sem_t1, sem_o0, sem_o1):
    tokbuf = (tok0, tok1)
    sem_t = (sem_t0, sem_t1)
    sem_o = (sem_o0, sem_o1)

    wid = lax.axis_index("s") * _NC + lax.axis_index("c")
    sbase = wid * _SPW

    for b in range(_B):
        pltpu.async_copy(seq_hbm.at[pl.ds(b * _S + sbase, _SPW)],
                         seqv.at[pl.ds(b * _SPW, _SPW)], sem_t0)
        pltpu.async_copy(
            sidrep_hbm.at[pl.ds((b * _S + sbase) * _LANES, _SPW * _LANES)],
            sidrv.at[pl.ds(b * _SPW * _LANES, _SPW * _LANES)], sem_t1)
    cp_pe = pltpu.async_copy(pe_hbm.at[pl.ds(sbase * _DP, _SPW * _DP)],
                             pebv, sem_o0)
    cp_seg = pltpu.async_copy(seg_hbm, segtab, sem_o1)
    for b in range(_B):
        pltpu.make_async_copy(seq_hbm.at[pl.ds(0, _SPW)],
                              seqv.at[pl.ds(0, _SPW)], sem_t0).wait()
        pltpu.make_async_copy(sidrep_hbm.at[pl.ds(0, _SPW * _LANES)],
                              sidrv.at[pl.ds(0, _SPW * _LANES)], sem_t1).wait()
    cp_pe.wait()
    cp_seg.wait()

    def chunk_pos(c):
        b = c // _CPB
        cl = lax.rem(c, _CPB)
        return b * _S + sbase + cl * _C, cl

    def issue(c, n):
        pltpu.async_copy(tok_hbm.at[seqv.at[pl.ds(c * _C, _C)]],
                         tokbuf[n], sem_t[n])

    def wait_gathers(n):
        pltpu.make_async_copy(tok_hbm.at[pl.ds(0, _C)], tokbuf[n],
                              sem_t[n]).wait()

    def compute(c, n):
        tv = tokbuf[n]
        jbase = c * (_C * _LANES)
        _, cl = chunk_pos(c)
        pchunk = cl * (_C * _DP)

        for kb in range(_NKB):
            d0 = kb * (_LANES * _KBLK)
            sg = [[segtab[j, pl.ds(d0 + q * _LANES, _LANES)]
                   for q in range(_KBLK)] for j in range(3)]

            @plsc.parallel_loop(0, _C, unroll=4)
            def _(r, d0=d0, sg=sg, pchunk=pchunk):
                jv = sidrv[pl.ds(jbase + r * _LANES, _LANES)]
                m1 = jv == 1
                m2 = jv == 2
                pebase = pl.multiple_of(pchunk + r * _DP, _LANES)
                for h in range(_KBLK // 2):
                    w = pebv[pl.ds(pebase + (d0 // 2) + h * _LANES, _LANES)]
                    pa = lax.bitcast_convert_type(lax.shift_left(w, 16),
                                                  jnp.float32)
                    pb = lax.bitcast_convert_type(w & jnp.int32(-65536),
                                                  jnp.float32)
                    for t, pe32 in ((0, pa), (1, pb)):
                        q = 2 * h + t
                        sl = pl.ds(d0 + q * _LANES, _LANES)
                        sgv = jnp.where(m1, sg[1][q], sg[0][q])
                        sgv = jnp.where(m2, sg[2][q], sgv)
                        tv[r, sl] = tv[r, sl] + pe32 + sgv

    def flush(c, n):
        row0, _ = chunk_pos(c)
        pltpu.async_copy(tokbuf[n], out_hbm.at[pl.ds(row0, _C)], sem_o[n])

    def wait_flush(n):
        pltpu.make_async_copy(tokbuf[n], out_hbm.at[pl.ds(0, _C)],
                              sem_o[n]).wait()

    issue(0, 0)

    def pair_body(i, _):
        c0 = 2 * i
        c1 = 2 * i + 1

        @pl.when(i > 0)
        def _():
            wait_flush(1)

        issue(c1, 1)
        wait_gathers(0)
        compute(c0, 0)
        flush(c0, 0)

        @pl.when(i + 1 < _NCH // 2)
        def _():
            wait_flush(0)
            issue(c0 + 2, 0)

        wait_gathers(1)
        compute(c1, 1)
        flush(c1, 1)
        return 0

    lax.fori_loop(0, _NCH // 2, pair_body, 0)
    wait_flush(0)
    wait_flush(1)


def kernel(sequence, segment_ids, token_table, segment_table):
    seq = sequence.reshape(_N).astype(jnp.int32)
    sidrep = jnp.repeat(segment_ids.reshape(_N).astype(jnp.int32), _LANES)
    pe = jnp.asarray(_PE_PACKED.reshape(-1))
    out = _embed(token_table.astype(jnp.float32),
                 segment_table.astype(jnp.float32), seq, sidrep, pe)
    return out.reshape(_B, _S, _D)


# speedup vs baseline: 1.0127x; 1.0127x over previous
import functools

import numpy as np
import jax
import jax.numpy as jnp
from jax import lax
from jax.experimental import pallas as pl
from jax.experimental.pallas import tpu as pltpu
from jax.experimental.pallas import tpu_sc as plsc

_D = 768
_B = 4
_S = 4096
_N = _B * _S
_NC = 2
_NS = 16
_NW = _NC * _NS
_NPW = _N // _NW
_C = 32
_NCH = _NPW // _C
_LANES = 16
_KBLK = 4
_NKB = _D // (_LANES * _KBLK)


def _build_pe(seq_len, d_model):
    pos = np.arange(seq_len, dtype=np.float32)[:, None]
    div = np.exp(np.arange(0, d_model, 2, dtype=np.float32)
                 * (-np.log(10000.0) / d_model))
    pe = np.zeros((seq_len, d_model), dtype=np.float32)
    pe[:, 0::2] = np.sin(pos * div)
    pe[:, 1::2] = np.cos(pos * div)
    return pe


def _pack_pe_bf16(pe):
    u = pe.view(np.uint32)
    rb = ((u.astype(np.uint64) + 0x7FFF + ((u >> 16) & 1)) >> 16).astype(np.uint32)
    g = rb.reshape(pe.shape[0], pe.shape[1] // 32, 2, 16)
    words = g[:, :, 0, :] | (g[:, :, 1, :] << 16)
    return words.reshape(pe.shape[0], pe.shape[1] // 2).view(np.int32)


_PE = _build_pe(_S, _D)
_PE_PACKED = _pack_pe_bf16(_PE)
_DP = _D // 2

_mesh = plsc.VectorSubcoreMesh(core_axis_name="c", subcore_axis_name="s")


@functools.partial(
    pl.kernel,
    mesh=_mesh,
    out_type=jax.ShapeDtypeStruct((_N, _D), jnp.float32),
    scratch_types=[
        pltpu.VMEM((_NPW,), jnp.int32),
        pltpu.VMEM((_NPW * _LANES,), jnp.int32),
        pltpu.VMEM((3, _D), jnp.float32),
        pltpu.VMEM((_C, _D), jnp.float32),
        pltpu.VMEM((_C, _D), jnp.float32),
        pltpu.VMEM((_C * _DP,), jnp.int32),
        pltpu.VMEM((_C * _DP,), jnp.int32),
        pltpu.SemaphoreType.DMA,
        pltpu.SemaphoreType.DMA,
        pltpu.SemaphoreType.DMA,
        pltpu.SemaphoreType.DMA,
        pltpu.SemaphoreType.DMA,
        pltpu.SemaphoreType.DMA,
    ],
)
def _embed(tok_hbm, seg_hbm, seq_hbm, sidrep_hbm, pe_hbm, out_hbm,
           seqv, sidrv, segtab, tok0, tok1, pe0, pe1,
           sem_t0, sem_t1, sem_p0, sem_p1, sem_o0, sem_o1):
    tokbuf = (tok0, tok1)
    pebuf = (pe0, pe1)
    sem_t = (sem_t0, sem_t1)
    sem_p = (sem_p0, sem_p1)
    sem_o = (sem_o0, sem_o1)

    wid = lax.axis_index("s") * _NC + lax.axis_index("c")
    base = wid * _NPW
    s0 = lax.rem(base, _S)

    cp_a = pltpu.async_copy(seq_hbm.at[pl.ds(base, _NPW)], seqv, sem_o0)
    cp_b = pltpu.async_copy(
        sidrep_hbm.at[pl.ds(base * _LANES, _NPW * _LANES)], sidrv, sem_o1)
    cp_c = pltpu.async_copy(seg_hbm, segtab, sem_p0)
    cp_a.wait()
    cp_b.wait()
    cp_c.wait()

    def issue(c, b):
        pltpu.async_copy(tok_hbm.at[seqv.at[pl.ds(c * _C, _C)]],
                         tokbuf[b], sem_t[b])
        pltpu.async_copy(pe_hbm.at[pl.ds((s0 + c * _C) * _DP, _C * _DP)],
                         pebuf[b], sem_p[b])

    def wait_gathers(b):
        pltpu.make_async_copy(tok_hbm.at[pl.ds(0, _C)], tokbuf[b],
                              sem_t[b]).wait()
        pltpu.make_async_copy(pe_hbm.at[pl.ds(0, _C * _DP)], pebuf[b],
                              sem_p[b]).wait()

    def compute(c, b):
        tv = tokbuf[b]
        pv = pebuf[b]
        jbase = c * (_C * _LANES)

        for kb in range(_NKB):
            d0 = kb * (_LANES * _KBLK)
            sg = [[segtab[j, pl.ds(d0 + q * _LANES, _LANES)] for q in range(_KBLK)]
                  for j in range(3)]

            @plsc.parallel_loop(0, _C, unroll=4)
            def _(r, d0=d0, sg=sg):
                jv = sidrv[pl.ds(jbase + r * _LANES, _LANES)]
                m1 = jv == 1
                m2 = jv == 2
                pebase = pl.multiple_of(r * _DP, _LANES)
                for h in range(_KBLK // 2):
                    w = pv[pl.ds(pebase + (d0 // 2) + h * _LANES, _LANES)]
                    pa = lax.bitcast_convert_type(lax.shift_left(w, 16),
                                                  jnp.float32)
                    pb = lax.bitcast_convert_type(w & jnp.int32(-65536),
                                                  jnp.float32)
                    for t, pe32 in ((0, pa), (1, pb)):
                        q = 2 * h + t
                        sl = pl.ds(d0 + q * _LANES, _LANES)
                        sgv = jnp.where(m1, sg[1][q], sg[0][q])
                        sgv = jnp.where(m2, sg[2][q], sgv)
                        tv[r, sl] = tv[r, sl] + pe32 + sgv

    def flush(c, b):
        pltpu.async_copy(tokbuf[b], out_hbm.at[pl.ds(base + c * _C, _C)],
                         sem_o[b])

    def wait_flush(b):
        pltpu.make_async_copy(tokbuf[b], out_hbm.at[pl.ds(0, _C)],
                              sem_o[b]).wait()

    issue(0, 0)

    def pair_body(i, _):
        c0 = 2 * i
        c1 = 2 * i + 1

        @pl.when(i > 0)
        def _():
            wait_flush(1)

        issue(c1, 1)
        wait_gathers(0)
        compute(c0, 0)
        flush(c0, 0)

        @pl.when(i + 1 < _NCH // 2)
        def _():
            wait_flush(0)
            issue(c0 + 2, 0)

        wait_gathers(1)
        compute(c1, 1)
        flush(c1, 1)
        return 0

    lax.fori_loop(0, _NCH // 2, pair_body, 0)
    wait_flush(0)
    wait_flush(1)


def kernel(sequence, segment_ids, token_table, segment_table):
    seq = sequence.reshape(_N).astype(jnp.int32)
    sidrep = jnp.repeat(segment_ids.reshape(_N).astype(jnp.int32), _LANES)
    pe = jnp.asarray(_PE_PACKED.reshape(-1))
    out = _embed(token_table.astype(jnp.float32),
                 segment_table.astype(jnp.float32), seq, sidrep, pe)
    return out.reshape(_B, _S, _D)
